# initial kernel scaffold (unmeasured)
import jax
import jax.numpy as jnp
from jax import lax
from jax.experimental import pallas as pl
from jax.experimental.pallas import tpu as pltpu


def kernel(
    x,
):
    def body(*refs):
        pass

    out_shape = jax.ShapeDtypeStruct(..., jnp.float32)
    return pl.pallas_call(body, out_shape=out_shape)(...)


# baseline (device time: 47757 ns/iter reference)
import jax
import jax.numpy as jnp
from jax import lax
from jax.experimental import pallas as pl
from jax.experimental.pallas import tpu as pltpu

N_DEV = 4
M = 1024
N = 1024
H = 256
Q = 128


def kernel(x):
    x2 = x.reshape(M, N)

    def body(x_ref, out_ref, bufA1, bufB1, bufA2, bufB2,
             sendA, recvA, sendB, recvB):
        p = lax.axis_index("i")
        yp = p ^ 1
        xp = 3 - p
        s1 = (p & 1) ^ (p >> 1)
        s2 = p >> 1
        s1b = p >> 1
        s2b = p & 1

        barrier = pltpu.get_barrier_semaphore()
        for nbr in (yp, xp):
            pl.semaphore_signal(
                barrier, inc=1,
                device_id=(nbr,), device_id_type=pl.DeviceIdType.MESH,
            )
        pl.semaphore_wait(barrier, 2)

        aK = H * s1
        aS = H * (1 - s1)
        bK = 512 + H * s1b
        bS = 512 + H * (1 - s1b)

        r1a = pltpu.make_async_remote_copy(
            src_ref=x_ref.at[pl.ds(aS, H), :], dst_ref=bufA1,
            send_sem=sendA.at[0], recv_sem=recvA.at[0],
            device_id=(yp,), device_id_type=pl.DeviceIdType.MESH)
        r1b = pltpu.make_async_remote_copy(
            src_ref=x_ref.at[pl.ds(bS, H), :], dst_ref=bufB1,
            send_sem=sendB.at[0], recv_sem=recvB.at[0],
            device_id=(xp,), device_id_type=pl.DeviceIdType.MESH)
        r1a.start()
        r1b.start()
        r1a.wait()
        r1b.wait()
        out_ref[pl.ds(aK, H), :] = x_ref[pl.ds(aK, H), :] + bufA1[:, :]
        out_ref[pl.ds(bK, H), :] = x_ref[pl.ds(bK, H), :] + bufB1[:, :]

        a2K = aK + Q * s2
        a2S = aK + Q * (1 - s2)
        b2K = bK + Q * s2b
        b2S = bK + Q * (1 - s2b)

        r2a = pltpu.make_async_remote_copy(
            src_ref=out_ref.at[pl.ds(a2S, Q), :], dst_ref=bufA2,
            send_sem=sendA.at[1], recv_sem=recvA.at[1],
            device_id=(xp,), device_id_type=pl.DeviceIdType.MESH)
        r2b = pltpu.make_async_remote_copy(
            src_ref=out_ref.at[pl.ds(b2S, Q), :], dst_ref=bufB2,
            send_sem=sendB.at[1], recv_sem=recvB.at[1],
            device_id=(yp,), device_id_type=pl.DeviceIdType.MESH)
        r2a.start()
        r2b.start()
        r2a.wait()
        r2b.wait()
        out_ref[pl.ds(a2K, Q), :] = out_ref[pl.ds(a2K, Q), :] + bufA2[:, :]
        out_ref[pl.ds(b2K, Q), :] = out_ref[pl.ds(b2K, Q), :] + bufB2[:, :]

        r3a = pltpu.make_async_remote_copy(
            src_ref=out_ref.at[pl.ds(a2K, Q), :],
            dst_ref=out_ref.at[pl.ds(a2K, Q), :],
            send_sem=sendA.at[2], recv_sem=recvA.at[2],
            device_id=(xp,), device_id_type=pl.DeviceIdType.MESH)
        r3b = pltpu.make_async_remote_copy(
            src_ref=out_ref.at[pl.ds(b2K, Q), :],
            dst_ref=out_ref.at[pl.ds(b2K, Q), :],
            send_sem=sendB.at[2], recv_sem=recvB.at[2],
            device_id=(yp,), device_id_type=pl.DeviceIdType.MESH)
        r3a.start()
        r3b.start()
        r3a.wait()
        r3b.wait()

        r4a = pltpu.make_async_remote_copy(
            src_ref=out_ref.at[pl.ds(aK, H), :],
            dst_ref=out_ref.at[pl.ds(aK, H), :],
            send_sem=sendA.at[3], recv_sem=recvA.at[3],
            device_id=(yp,), device_id_type=pl.DeviceIdType.MESH)
        r4b = pltpu.make_async_remote_copy(
            src_ref=out_ref.at[pl.ds(bK, H), :],
            dst_ref=out_ref.at[pl.ds(bK, H), :],
            send_sem=sendB.at[3], recv_sem=recvB.at[3],
            device_id=(xp,), device_id_type=pl.DeviceIdType.MESH)
        r4a.start()
        r4b.start()
        r4a.wait()
        r4b.wait()

    return pl.pallas_call(
        body,
        out_shape=jax.ShapeDtypeStruct((M, N), jnp.float32),
        in_specs=[pl.BlockSpec(memory_space=pltpu.VMEM)],
        out_specs=pl.BlockSpec(memory_space=pltpu.VMEM),
        scratch_shapes=[
            pltpu.VMEM((H, N), jnp.float32),
            pltpu.VMEM((H, N), jnp.float32),
            pltpu.VMEM((Q, N), jnp.float32),
            pltpu.VMEM((Q, N), jnp.float32),
            pltpu.SemaphoreType.DMA((4,)),
            pltpu.SemaphoreType.DMA((4,)),
            pltpu.SemaphoreType.DMA((4,)),
            pltpu.SemaphoreType.DMA((4,)),
        ],
        compiler_params=pltpu.CompilerParams(collective_id=0),
    )(x2)


# device time: 46218 ns/iter; 1.0333x vs baseline; 1.0333x over previous
import jax
import jax.numpy as jnp
from jax import lax
from jax.experimental import pallas as pl
from jax.experimental.pallas import tpu as pltpu

N_DEV = 4
M = 1024
N = 1024
H = 256
Q = 128


def kernel(x):
    x2 = x.reshape(M, N)

    def body(x_ref, out_ref, bufA1, bufB1, bufA2, bufB2,
             sendA, recvA, sendB, recvB):
        p = lax.axis_index("i")
        yp = p ^ 1
        xp = 3 - p
        s1 = (p & 1) ^ (p >> 1)
        s2 = p >> 1
        s1b = p >> 1
        s2b = p & 1

        barrier = pltpu.get_barrier_semaphore()
        for nbr in (yp, xp):
            pl.semaphore_signal(
                barrier, inc=1,
                device_id=(nbr,), device_id_type=pl.DeviceIdType.MESH,
            )
        pl.semaphore_wait(barrier, 2)

        aK = H * s1
        aS = H * (1 - s1)
        bK = 512 + H * s1b
        bS = 512 + H * (1 - s1b)
        a2K = aK + Q * s2
        a2S = aK + Q * (1 - s2)
        b2K = bK + Q * s2b
        b2S = bK + Q * (1 - s2b)
        aFwd = Q * (1 - s2)
        aKeep = Q * s2
        bFwd = Q * (1 - s2b)
        bKeep = Q * s2b

        def rdma(src, dst, ssem, rsem, tgt):
            return pltpu.make_async_remote_copy(
                src_ref=src, dst_ref=dst, send_sem=ssem, recv_sem=rsem,
                device_id=(tgt,), device_id_type=pl.DeviceIdType.MESH)

        r1a1 = rdma(x_ref.at[pl.ds(aS + aFwd, Q), :],
                    bufA1.at[pl.ds(aFwd, Q), :],
                    sendA.at[0], recvA.at[0], yp)
        r1b1 = rdma(x_ref.at[pl.ds(bS + bKeep, Q), :],
                    bufB1.at[pl.ds(bKeep, Q), :],
                    sendB.at[0], recvB.at[0], xp)
        r1a2 = rdma(x_ref.at[pl.ds(aS + aKeep, Q), :],
                    bufA1.at[pl.ds(aKeep, Q), :],
                    sendA.at[1], recvA.at[1], yp)
        r1b2 = rdma(x_ref.at[pl.ds(bS + bFwd, Q), :],
                    bufB1.at[pl.ds(bFwd, Q), :],
                    sendB.at[1], recvB.at[1], xp)
        r1a1.start()
        r1b1.start()
        r1a2.start()
        r1b2.start()

        r1a1.wait_recv()
        out_ref[pl.ds(a2S, Q), :] = x_ref[pl.ds(a2S, Q), :] + \
            bufA1[pl.ds(aFwd, Q), :]
        r2a = rdma(out_ref.at[pl.ds(a2S, Q), :], bufA2,
                   sendA.at[2], recvA.at[2], xp)
        r2a.start()
        r1b1.wait_recv()
        out_ref[pl.ds(b2S, Q), :] = x_ref[pl.ds(b2S, Q), :] + \
            bufB1[pl.ds(bFwd, Q), :]
        r2b = rdma(out_ref.at[pl.ds(b2S, Q), :], bufB2,
                   sendB.at[2], recvB.at[2], yp)
        r2b.start()

        r1a2.wait_recv()
        out_ref[pl.ds(a2K, Q), :] = x_ref[pl.ds(a2K, Q), :] + \
            bufA1[pl.ds(aKeep, Q), :]
        r1b2.wait_recv()
        out_ref[pl.ds(b2K, Q), :] = x_ref[pl.ds(b2K, Q), :] + \
            bufB1[pl.ds(bKeep, Q), :]

        r2a.wait_recv()
        out_ref[pl.ds(a2K, Q), :] = out_ref[pl.ds(a2K, Q), :] + bufA2[:, :]
        r3a = rdma(out_ref.at[pl.ds(a2K, Q), :],
                   out_ref.at[pl.ds(a2K, Q), :],
                   sendA.at[3], recvA.at[3], xp)
        r3a.start()
        r4a1 = rdma(out_ref.at[pl.ds(a2K, Q), :],
                    out_ref.at[pl.ds(a2K, Q), :],
                    sendA.at[4], recvA.at[4], yp)
        r4a1.start()
        r2b.wait_recv()
        out_ref[pl.ds(b2K, Q), :] = out_ref[pl.ds(b2K, Q), :] + bufB2[:, :]
        r3b = rdma(out_ref.at[pl.ds(b2K, Q), :],
                   out_ref.at[pl.ds(b2K, Q), :],
                   sendB.at[3], recvB.at[3], yp)
        r3b.start()
        r4b1 = rdma(out_ref.at[pl.ds(b2K, Q), :],
                    out_ref.at[pl.ds(b2K, Q), :],
                    sendB.at[4], recvB.at[4], xp)
        r4b1.start()

        r3a.wait_recv()
        r4a2 = rdma(out_ref.at[pl.ds(a2S, Q), :],
                    out_ref.at[pl.ds(a2S, Q), :],
                    sendA.at[5], recvA.at[5], yp)
        r4a2.start()
        r3b.wait_recv()
        r4b2 = rdma(out_ref.at[pl.ds(b2S, Q), :],
                    out_ref.at[pl.ds(b2S, Q), :],
                    sendB.at[5], recvB.at[5], xp)
        r4b2.start()

        r4a1.wait_recv()
        r4a2.wait_recv()
        r4b1.wait_recv()
        r4b2.wait_recv()

        for r in (r1a1, r1b1, r1a2, r1b2, r2a, r2b,
                  r3a, r3b, r4a1, r4b1, r4a2, r4b2):
            r.wait_send()

    return pl.pallas_call(
        body,
        out_shape=jax.ShapeDtypeStruct((M, N), jnp.float32),
        in_specs=[pl.BlockSpec(memory_space=pltpu.VMEM)],
        out_specs=pl.BlockSpec(memory_space=pltpu.VMEM),
        scratch_shapes=[
            pltpu.VMEM((H, N), jnp.float32),
            pltpu.VMEM((H, N), jnp.float32),
            pltpu.VMEM((Q, N), jnp.float32),
            pltpu.VMEM((Q, N), jnp.float32),
            pltpu.SemaphoreType.DMA((6,)),
            pltpu.SemaphoreType.DMA((6,)),
            pltpu.SemaphoreType.DMA((6,)),
            pltpu.SemaphoreType.DMA((6,)),
        ],
        compiler_params=pltpu.CompilerParams(collective_id=0),
    )(x2)


# device time: 43074 ns/iter; 1.1087x vs baseline; 1.0730x over previous
import jax
import jax.numpy as jnp
from jax import lax
from jax.experimental import pallas as pl
from jax.experimental.pallas import tpu as pltpu

N_DEV = 4
M = 1024
N = 1024
H = 256
Q = 128
CW = 512
NC = 2


def kernel(x):
    x2 = x.reshape(M, N)

    def body(x_ref, out_ref, bufA1, bufB1, bufA2, bufB2,
             sendA, recvA, sendB, recvB):
        p = lax.axis_index("i")
        yp = p ^ 1
        xp = 3 - p
        s1 = (p & 1) ^ (p >> 1)
        s2 = p >> 1
        s1b = p >> 1
        s2b = p & 1

        barrier = pltpu.get_barrier_semaphore()
        for nbr in (yp, xp):
            pl.semaphore_signal(
                barrier, inc=1,
                device_id=(nbr,), device_id_type=pl.DeviceIdType.MESH,
            )
        pl.semaphore_wait(barrier, 2)

        aK = H * s1
        aS = H * (1 - s1)
        bK = 512 + H * s1b
        bS = 512 + H * (1 - s1b)
        a2K = aK + Q * s2
        a2S = aK + Q * (1 - s2)
        b2K = bK + Q * s2b
        b2S = bK + Q * (1 - s2b)
        aFwd = Q * (1 - s2)
        aKeep = Q * s2
        bFwd = Q * (1 - s2b)
        bKeep = Q * s2b

        def rdma(src, dst, ssem, rsem, tgt):
            return pltpu.make_async_remote_copy(
                src_ref=src, dst_ref=dst, send_sem=ssem, recv_sem=rsem,
                device_id=(tgt,), device_id_type=pl.DeviceIdType.MESH)

        def make_col(c):
            cs = pl.ds(c * CW, CW)
            o = 6 * c
            d = {}
            d["r1a1"] = rdma(x_ref.at[pl.ds(aS + aFwd, Q), cs],
                             bufA1.at[pl.ds(aFwd, Q), cs],
                             sendA.at[o + 0], recvA.at[o + 0], yp)
            d["r1b1"] = rdma(x_ref.at[pl.ds(bS + bKeep, Q), cs],
                             bufB1.at[pl.ds(bKeep, Q), cs],
                             sendB.at[o + 0], recvB.at[o + 0], xp)
            d["r1a2"] = rdma(x_ref.at[pl.ds(aS + aKeep, Q), cs],
                             bufA1.at[pl.ds(aKeep, Q), cs],
                             sendA.at[o + 1], recvA.at[o + 1], yp)
            d["r1b2"] = rdma(x_ref.at[pl.ds(bS + bFwd, Q), cs],
                             bufB1.at[pl.ds(bFwd, Q), cs],
                             sendB.at[o + 1], recvB.at[o + 1], xp)
            d["r2a"] = rdma(out_ref.at[pl.ds(a2S, Q), cs],
                            bufA2.at[:, cs],
                            sendA.at[o + 2], recvA.at[o + 2], xp)
            d["r2b"] = rdma(out_ref.at[pl.ds(b2S, Q), cs],
                            bufB2.at[:, cs],
                            sendB.at[o + 2], recvB.at[o + 2], yp)
            d["r3a"] = rdma(out_ref.at[pl.ds(a2K, Q), cs],
                            out_ref.at[pl.ds(a2K, Q), cs],
                            sendA.at[o + 3], recvA.at[o + 3], xp)
            d["r3b"] = rdma(out_ref.at[pl.ds(b2K, Q), cs],
                            out_ref.at[pl.ds(b2K, Q), cs],
                            sendB.at[o + 3], recvB.at[o + 3], yp)
            d["r4a1"] = rdma(out_ref.at[pl.ds(a2K, Q), cs],
                             out_ref.at[pl.ds(a2K, Q), cs],
                             sendA.at[o + 4], recvA.at[o + 4], yp)
            d["r4b1"] = rdma(out_ref.at[pl.ds(b2K, Q), cs],
                             out_ref.at[pl.ds(b2K, Q), cs],
                             sendB.at[o + 4], recvB.at[o + 4], xp)
            d["r4a2"] = rdma(out_ref.at[pl.ds(a2S, Q), cs],
                             out_ref.at[pl.ds(a2S, Q), cs],
                             sendA.at[o + 5], recvA.at[o + 5], yp)
            d["r4b2"] = rdma(out_ref.at[pl.ds(b2S, Q), cs],
                             out_ref.at[pl.ds(b2S, Q), cs],
                             sendB.at[o + 5], recvB.at[o + 5], xp)
            return d

        cols = [make_col(0), make_col(1)]

        def cslice(c):
            return pl.ds(c * CW, CW)

        for c in range(NC):
            cols[c]["r1a1"].start()
            cols[c]["r1b1"].start()
        for c in range(NC):
            cols[c]["r1a2"].start()
            cols[c]["r1b2"].start()

        for c in range(NC):
            cs = cslice(c)
            cols[c]["r1a1"].wait_recv()
            out_ref[pl.ds(a2S, Q), cs] = x_ref[pl.ds(a2S, Q), cs] + \
                bufA1[pl.ds(aFwd, Q), cs]
            cols[c]["r2a"].start()
            cols[c]["r1b1"].wait_recv()
            out_ref[pl.ds(b2S, Q), cs] = x_ref[pl.ds(b2S, Q), cs] + \
                bufB1[pl.ds(bFwd, Q), cs]
            cols[c]["r2b"].start()

        for c in range(NC):
            cs = cslice(c)
            cols[c]["r1a2"].wait_recv()
            out_ref[pl.ds(a2K, Q), cs] = x_ref[pl.ds(a2K, Q), cs] + \
                bufA1[pl.ds(aKeep, Q), cs]
            cols[c]["r1b2"].wait_recv()
            out_ref[pl.ds(b2K, Q), cs] = x_ref[pl.ds(b2K, Q), cs] + \
                bufB1[pl.ds(bKeep, Q), cs]

        for c in range(NC):
            cs = cslice(c)
            cols[c]["r2a"].wait_recv()
            out_ref[pl.ds(a2K, Q), cs] = out_ref[pl.ds(a2K, Q), cs] + \
                bufA2[:, cs]
            cols[c]["r3a"].start()
            cols[c]["r4a1"].start()
            cols[c]["r2b"].wait_recv()
            out_ref[pl.ds(b2K, Q), cs] = out_ref[pl.ds(b2K, Q), cs] + \
                bufB2[:, cs]
            cols[c]["r3b"].start()
            cols[c]["r4b1"].start()

        for c in range(NC):
            cols[c]["r3a"].wait_recv()
            cols[c]["r4a2"].start()
            cols[c]["r3b"].wait_recv()
            cols[c]["r4b2"].start()

        for c in range(NC):
            cols[c]["r4a1"].wait_recv()
            cols[c]["r4a2"].wait_recv()
            cols[c]["r4b1"].wait_recv()
            cols[c]["r4b2"].wait_recv()

        for c in range(NC):
            for r in cols[c].values():
                r.wait_send()

    return pl.pallas_call(
        body,
        out_shape=jax.ShapeDtypeStruct((M, N), jnp.float32),
        in_specs=[pl.BlockSpec(memory_space=pltpu.VMEM)],
        out_specs=pl.BlockSpec(memory_space=pltpu.VMEM),
        scratch_shapes=[
            pltpu.VMEM((H, N), jnp.float32),
            pltpu.VMEM((H, N), jnp.float32),
            pltpu.VMEM((Q, N), jnp.float32),
            pltpu.VMEM((Q, N), jnp.float32),
            pltpu.SemaphoreType.DMA((12,)),
            pltpu.SemaphoreType.DMA((12,)),
            pltpu.SemaphoreType.DMA((12,)),
            pltpu.SemaphoreType.DMA((12,)),
        ],
        compiler_params=pltpu.CompilerParams(collective_id=0),
    )(x2)


# device time: 42424 ns/iter; 1.1257x vs baseline; 1.0153x over previous
import jax
import jax.numpy as jnp
from jax import lax
from jax.experimental import pallas as pl
from jax.experimental.pallas import tpu as pltpu

N_DEV = 4
M = 1024
N = 1024
H = 256
Q = 128
CW = 256
NC = 4


def kernel(x):
    x2 = x.reshape(M, N)

    def body(x_ref, out_ref, bufA1, bufB1, bufA2, bufB2,
             sendA, recvA, sendB, recvB):
        p = lax.axis_index("i")
        yp = p ^ 1
        xp = 3 - p
        s1 = (p & 1) ^ (p >> 1)
        s2 = p >> 1
        s1b = p >> 1
        s2b = p & 1

        barrier = pltpu.get_barrier_semaphore()
        for nbr in (yp, xp):
            pl.semaphore_signal(
                barrier, inc=1,
                device_id=(nbr,), device_id_type=pl.DeviceIdType.MESH,
            )
        pl.semaphore_wait(barrier, 2)

        aK = H * s1
        aS = H * (1 - s1)
        bK = 512 + H * s1b
        bS = 512 + H * (1 - s1b)
        a2K = aK + Q * s2
        a2S = aK + Q * (1 - s2)
        b2K = bK + Q * s2b
        b2S = bK + Q * (1 - s2b)
        aFwd = Q * (1 - s2)
        aKeep = Q * s2
        bFwd = Q * (1 - s2b)
        bKeep = Q * s2b

        def rdma(src, dst, ssem, rsem, tgt):
            return pltpu.make_async_remote_copy(
                src_ref=src, dst_ref=dst, send_sem=ssem, recv_sem=rsem,
                device_id=(tgt,), device_id_type=pl.DeviceIdType.MESH)

        def make_col(c):
            cs = pl.ds(c * CW, CW)
            o = 6 * c
            d = {}
            d["r1a1"] = rdma(x_ref.at[pl.ds(aS + aFwd, Q), cs],
                             bufA1.at[pl.ds(aFwd, Q), cs],
                             sendA.at[o + 0], recvA.at[o + 0], yp)
            d["r1b1"] = rdma(x_ref.at[pl.ds(bS + bKeep, Q), cs],
                             bufB1.at[pl.ds(bKeep, Q), cs],
                             sendB.at[o + 0], recvB.at[o + 0], xp)
            d["r1a2"] = rdma(x_ref.at[pl.ds(aS + aKeep, Q), cs],
                             bufA1.at[pl.ds(aKeep, Q), cs],
                             sendA.at[o + 1], recvA.at[o + 1], yp)
            d["r1b2"] = rdma(x_ref.at[pl.ds(bS + bFwd, Q), cs],
                             bufB1.at[pl.ds(bFwd, Q), cs],
                             sendB.at[o + 1], recvB.at[o + 1], xp)
            d["r2a"] = rdma(out_ref.at[pl.ds(a2S, Q), cs],
                            bufA2.at[:, cs],
                            sendA.at[o + 2], recvA.at[o + 2], xp)
            d["r2b"] = rdma(out_ref.at[pl.ds(b2S, Q), cs],
                            bufB2.at[:, cs],
                            sendB.at[o + 2], recvB.at[o + 2], yp)
            d["r3a"] = rdma(out_ref.at[pl.ds(a2K, Q), cs],
                            out_ref.at[pl.ds(a2K, Q), cs],
                            sendA.at[o + 3], recvA.at[o + 3], xp)
            d["r3b"] = rdma(out_ref.at[pl.ds(b2K, Q), cs],
                            out_ref.at[pl.ds(b2K, Q), cs],
                            sendB.at[o + 3], recvB.at[o + 3], yp)
            d["r4a1"] = rdma(out_ref.at[pl.ds(a2K, Q), cs],
                             out_ref.at[pl.ds(a2K, Q), cs],
                             sendA.at[o + 4], recvA.at[o + 4], yp)
            d["r4b1"] = rdma(out_ref.at[pl.ds(b2K, Q), cs],
                             out_ref.at[pl.ds(b2K, Q), cs],
                             sendB.at[o + 4], recvB.at[o + 4], xp)
            d["r4a2"] = rdma(out_ref.at[pl.ds(a2S, Q), cs],
                             out_ref.at[pl.ds(a2S, Q), cs],
                             sendA.at[o + 5], recvA.at[o + 5], yp)
            d["r4b2"] = rdma(out_ref.at[pl.ds(b2S, Q), cs],
                             out_ref.at[pl.ds(b2S, Q), cs],
                             sendB.at[o + 5], recvB.at[o + 5], xp)
            return d

        cols = [make_col(c) for c in range(NC)]

        def cslice(c):
            return pl.ds(c * CW, CW)

        for c in range(NC):
            cols[c]["r1a1"].start()
            cols[c]["r1b1"].start()
        for c in range(NC):
            cols[c]["r1a2"].start()
            cols[c]["r1b2"].start()

        for c in range(NC):
            cs = cslice(c)
            cols[c]["r1a1"].wait_recv()
            out_ref[pl.ds(a2S, Q), cs] = x_ref[pl.ds(a2S, Q), cs] + \
                bufA1[pl.ds(aFwd, Q), cs]
            cols[c]["r2a"].start()
            cols[c]["r1b1"].wait_recv()
            out_ref[pl.ds(b2S, Q), cs] = x_ref[pl.ds(b2S, Q), cs] + \
                bufB1[pl.ds(bFwd, Q), cs]
            cols[c]["r2b"].start()

        for c in range(NC):
            cs = cslice(c)
            cols[c]["r1a2"].wait_recv()
            out_ref[pl.ds(a2K, Q), cs] = x_ref[pl.ds(a2K, Q), cs] + \
                bufA1[pl.ds(aKeep, Q), cs]
            cols[c]["r1b2"].wait_recv()
            out_ref[pl.ds(b2K, Q), cs] = x_ref[pl.ds(b2K, Q), cs] + \
                bufB1[pl.ds(bKeep, Q), cs]

        for c in range(NC):
            cs = cslice(c)
            cols[c]["r2a"].wait_recv()
            out_ref[pl.ds(a2K, Q), cs] = out_ref[pl.ds(a2K, Q), cs] + \
                bufA2[:, cs]
            cols[c]["r3a"].start()
            cols[c]["r4a1"].start()
            cols[c]["r2b"].wait_recv()
            out_ref[pl.ds(b2K, Q), cs] = out_ref[pl.ds(b2K, Q), cs] + \
                bufB2[:, cs]
            cols[c]["r3b"].start()
            cols[c]["r4b1"].start()

        for c in range(NC):
            cols[c]["r3a"].wait_recv()
            cols[c]["r4a2"].start()
            cols[c]["r3b"].wait_recv()
            cols[c]["r4b2"].start()

        for c in range(NC):
            cols[c]["r4a1"].wait_recv()
            cols[c]["r4a2"].wait_recv()
            cols[c]["r4b1"].wait_recv()
            cols[c]["r4b2"].wait_recv()

        for c in range(NC):
            for r in cols[c].values():
                r.wait_send()

    return pl.pallas_call(
        body,
        out_shape=jax.ShapeDtypeStruct((M, N), jnp.float32),
        in_specs=[pl.BlockSpec(memory_space=pltpu.VMEM)],
        out_specs=pl.BlockSpec(memory_space=pltpu.VMEM),
        scratch_shapes=[
            pltpu.VMEM((H, N), jnp.float32),
            pltpu.VMEM((H, N), jnp.float32),
            pltpu.VMEM((Q, N), jnp.float32),
            pltpu.VMEM((Q, N), jnp.float32),
            pltpu.SemaphoreType.DMA((24,)),
            pltpu.SemaphoreType.DMA((24,)),
            pltpu.SemaphoreType.DMA((24,)),
            pltpu.SemaphoreType.DMA((24,)),
        ],
        compiler_params=pltpu.CompilerParams(collective_id=0),
    )(x2)
